# baseline (speedup 1.0000x reference)
import functools

import jax
import jax.numpy as jnp
from jax import lax
from jax.experimental import pallas as pl
from jax.experimental.pallas import tpu as pltpu
from jax.experimental.pallas import tpu_sc as plsc

EMBED = 128
CCH = 64
NCB = 6
LANES = 16


def _make_sc_gather_cf(batch, dtype):
    info = plsc.get_sparse_core_info()
    nc, ns = info.num_cores, info.num_subcores
    nw = nc * ns
    assert batch % (nw * CCH) == 0, (batch, nw)
    kc = batch // (nw * CCH)
    rows = kc * CCH
    nit = 2 * kc
    mesh = plsc.VectorSubcoreMesh(core_axis_name="c", subcore_axis_name="s")

    out_t = (
        jax.ShapeDtypeStruct((batch, EMBED), dtype),
        jax.ShapeDtypeStruct((batch, EMBED), dtype),
        jax.ShapeDtypeStruct((batch,), dtype),
    )

    scratch = (
        [pltpu.VMEM((rows,), jnp.int32),
         pltpu.VMEM((rows,), jnp.int32)]
        + [pltpu.VMEM((CCH, EMBED), dtype) for _ in range(4)]
        + [pltpu.VMEM((CCH, EMBED), dtype) for _ in range(NCB)]
        + [pltpu.VMEM((rows,), dtype),
           pltpu.VMEM((LANES * (LANES + 1),), dtype)]
        + [pltpu.SemaphoreType.DMA] * (4 + 2 * NCB + 1)
    )

    @functools.partial(
        pl.kernel,
        out_type=out_t,
        mesh=mesh,
        compiler_params=pltpu.CompilerParams(needs_layout_passes=False),
        scratch_types=scratch,
    )
    def sc_kernel(uidx_hbm, iidx_hbm, ue_hbm, ie_hbm, uc_hbm, ic_hbm,
                  out_uc, out_ic, out_cf, *scr):
        idx_u, idx_i = scr[0], scr[1]
        ue_slots, ie_slots = (scr[2], scr[3]), (scr[4], scr[5])
        cbufs = scr[6:6 + NCB]
        cfv, tv = scr[6 + NCB], scr[7 + NCB]
        sems = scr[8 + NCB:]
        usems, isems = (sems[0], sems[1]), (sems[2], sems[3])
        cgsems = sems[4:4 + NCB]
        cssems = sems[4 + NCB:4 + 2 * NCB]
        cfsem = sems[4 + 2 * NCB]

        wid = lax.axis_index("s") * nc + lax.axis_index("c")
        pltpu.sync_copy(uidx_hbm.at[pl.ds(wid * rows, rows)], idx_u)
        pltpu.sync_copy(iidx_hbm.at[pl.ds(wid * rows, rows)], idx_i)

        def cf_gather(j):
            b = j % 2
            sl = pl.ds(j * CCH, CCH)
            return (
                pltpu.async_copy(ue_hbm.at[idx_u.at[sl]], ue_slots[b], usems[b]),
                pltpu.async_copy(ie_hbm.at[idx_i.at[sl]], ie_slots[b], isems[b]),
            )

        def item(i):
            j, tbl = i // 2, i % 2
            table = uc_hbm if tbl == 0 else ic_hbm
            idxv = idx_u if tbl == 0 else idx_i
            out = out_uc if tbl == 0 else out_ic
            return table, idxv, j, out

        def cgather(i):
            table, idxv, j, _ = item(i)
            b = i % NCB
            return pltpu.async_copy(
                table.at[idxv.at[pl.ds(j * CCH, CCH)]], cbufs[b], cgsems[b])

        def cstore(i):
            _, _, j, out = item(i)
            b = i % NCB
            return pltpu.async_copy(
                cbufs[b], out.at[pl.ds(wid * rows + j * CCH, CCH)], cssems[b])

        lane = lax.iota(jnp.int32, LANES)
        tpose = LANES + 1

        def cf_chunk(j):
            ur, ir = ue_slots[j % 2], ie_slots[j % 2]

            def group(g, _):
                for s16 in range(LANES):
                    s = g * LANES + s16
                    acc = ur[s, pl.ds(0, LANES)] * ir[s, pl.ds(0, LANES)]
                    for k in range(1, EMBED // LANES):
                        acc = acc + (ur[s, pl.ds(k * LANES, LANES)]
                                     * ir[s, pl.ds(k * LANES, LANES)])
                    plsc.store_scatter(tv, [lane * tpose + s16], acc)
                red = tv[pl.ds(0, LANES)]
                for l in range(1, LANES):
                    red = red + tv[pl.ds(l * tpose, LANES)]
                plsc.store_scatter(cfv, [j * CCH + g * LANES + lane], red)
                return 0

            lax.fori_loop(0, CCH // LANES, group, 0)

        cf_cp = cf_gather(0)
        g_cp = [None] * NCB
        s_cp = [None] * NCB
        for i in range(min(NCB - 2, nit)):
            g_cp[i % NCB] = cgather(i)
        for j in range(kc):
            nxt = cf_gather(j + 1) if j + 1 < kc else None
            for i in (2 * j, 2 * j + 1):
                g_cp[i % NCB].wait()
                s_cp[i % NCB] = cstore(i)
                ipre = i - 2
                inew = i + NCB - 2
                if inew < nit:
                    if ipre >= 0:
                        s_cp[ipre % NCB].wait()
                    g_cp[inew % NCB] = cgather(inew)
            cf_cp[0].wait()
            cf_cp[1].wait()
            cf_chunk(j)
            cf_cp = nxt
        cf_store = pltpu.async_copy(
            cfv, out_cf.at[pl.ds(wid * rows, rows)], cfsem)
        for i in range(max(0, nit - NCB), nit):
            s_cp[i % NCB].wait()
        cf_store.wait()

    return sc_kernel


def _tc_body(cf_ref, uc_ref, ic_ref, w1a_ref, w1b_ref, b1_ref,
             w2_ref, b2_ref, out_ref):
    h = jnp.dot(uc_ref[...], w1a_ref[...], preferred_element_type=jnp.float32)
    h = h + jnp.dot(ic_ref[...], w1b_ref[...],
                    preferred_element_type=jnp.float32)
    h = jnp.maximum(h + b1_ref[...], 0.0)
    out = jnp.dot(h, w2_ref[...], preferred_element_type=jnp.float32)
    out_ref[...] = cf_ref[...][:, None] + out + b2_ref[...]


def _tc_mlp(cf, uc, ic, W1, b1, w2, b2, blk):
    batch = uc.shape[0]
    hid = w2.shape[0]
    grid = (batch // blk,)
    row_spec = pl.BlockSpec((blk, EMBED), lambda i: (i, 0))
    full = lambda shape: pl.BlockSpec(shape, lambda i: (0,) * len(shape))
    return pl.pallas_call(
        _tc_body,
        grid=grid,
        in_specs=[
            pl.BlockSpec((blk,), lambda i: (i,)),
            row_spec, row_spec,
            pl.BlockSpec((EMBED, hid), lambda i: (0, 0)),
            pl.BlockSpec((EMBED, hid), lambda i: (1, 0)),
            full((1, hid)),
            full((hid, EMBED)), full((1, EMBED)),
        ],
        out_specs=row_spec,
        out_shape=jax.ShapeDtypeStruct((batch, EMBED), jnp.float32),
    )(cf, uc, ic, W1, W1, b1, w2, b2)


def kernel(user_indices, item_indices, user_emb, item_emb,
           user_content_emb, item_content_emb, W1, b1, W2, b2):
    uidx = user_indices.astype(jnp.int32)
    iidx = item_indices.astype(jnp.int32)
    batch = uidx.shape[0]
    hid = W2.shape[0]
    b1r, b2r = b1.reshape(1, hid), b2.reshape(1, EMBED)

    nh = 2
    bh = batch // nh
    sc = _make_sc_gather_cf(bh, user_emb.dtype)
    outs = []
    for h in range(nh):
        sl = slice(h * bh, (h + 1) * bh)
        uc_g, ic_g, cf = sc(uidx[sl], iidx[sl], user_emb, item_emb,
                            user_content_emb, item_content_emb)
        outs.append(_tc_mlp(cf, uc_g, ic_g, W1, b1r, W2, b2r, blk=2048))
    return jnp.concatenate(outs, axis=0)


# speedup vs baseline: 1.1173x; 1.1173x over previous
import functools

import jax
import jax.numpy as jnp
from jax import lax
from jax.experimental import pallas as pl
from jax.experimental.pallas import tpu as pltpu
from jax.experimental.pallas import tpu_sc as plsc

EMBED = 128
CCH = 64
NCB = 6
LANES = 16


def _make_sc_gather_cf(batch, dtype):
    info = plsc.get_sparse_core_info()
    nc, ns = info.num_cores, info.num_subcores
    nw = nc * ns
    assert batch % (nw * CCH) == 0, (batch, nw)
    kc = batch // (nw * CCH)
    rows = kc * CCH
    nit = 2 * kc
    mesh = plsc.VectorSubcoreMesh(core_axis_name="c", subcore_axis_name="s")

    out_t = (
        jax.ShapeDtypeStruct((batch, EMBED), dtype),
        jax.ShapeDtypeStruct((batch, EMBED), dtype),
        jax.ShapeDtypeStruct((batch,), dtype),
    )

    scratch = (
        [pltpu.VMEM((rows,), jnp.int32),
         pltpu.VMEM((rows,), jnp.int32)]
        + [pltpu.VMEM((CCH, EMBED), dtype) for _ in range(4)]
        + [pltpu.VMEM((CCH, EMBED), dtype) for _ in range(NCB)]
        + [pltpu.VMEM((rows,), dtype),
           pltpu.VMEM((LANES * (LANES + 1),), dtype)]
        + [pltpu.SemaphoreType.DMA] * (4 + 2 * NCB + 1)
    )

    @functools.partial(
        pl.kernel,
        out_type=out_t,
        mesh=mesh,
        compiler_params=pltpu.CompilerParams(needs_layout_passes=False),
        scratch_types=scratch,
    )
    def sc_kernel(uidx_hbm, iidx_hbm, ue_hbm, ie_hbm, uc_hbm, ic_hbm,
                  out_uc, out_ic, out_cf, *scr):
        idx_u, idx_i = scr[0], scr[1]
        ue_slots, ie_slots = (scr[2], scr[3]), (scr[4], scr[5])
        cbufs = scr[6:6 + NCB]
        cfv, tv = scr[6 + NCB], scr[7 + NCB]
        sems = scr[8 + NCB:]
        usems, isems = (sems[0], sems[1]), (sems[2], sems[3])
        cgsems = sems[4:4 + NCB]
        cssems = sems[4 + NCB:4 + 2 * NCB]
        cfsem = sems[4 + 2 * NCB]

        wid = lax.axis_index("s") * nc + lax.axis_index("c")
        pltpu.sync_copy(uidx_hbm.at[pl.ds(wid * rows, rows)], idx_u)
        pltpu.sync_copy(iidx_hbm.at[pl.ds(wid * rows, rows)], idx_i)

        def cf_gather(j):
            b = j % 2
            sl = pl.ds(j * CCH, CCH)
            return (
                pltpu.async_copy(ue_hbm.at[idx_u.at[sl]], ue_slots[b], usems[b]),
                pltpu.async_copy(ie_hbm.at[idx_i.at[sl]], ie_slots[b], isems[b]),
            )

        def item(i):
            j, tbl = i // 2, i % 2
            table = uc_hbm if tbl == 0 else ic_hbm
            idxv = idx_u if tbl == 0 else idx_i
            out = out_uc if tbl == 0 else out_ic
            return table, idxv, j, out

        def cgather(i):
            table, idxv, j, _ = item(i)
            b = i % NCB
            return pltpu.async_copy(
                table.at[idxv.at[pl.ds(j * CCH, CCH)]], cbufs[b], cgsems[b])

        def cstore(i):
            _, _, j, out = item(i)
            b = i % NCB
            return pltpu.async_copy(
                cbufs[b], out.at[pl.ds(wid * rows + j * CCH, CCH)], cssems[b])

        lane = lax.iota(jnp.int32, LANES)
        tpose = LANES + 1

        def cf_chunk(j):
            ur, ir = ue_slots[j % 2], ie_slots[j % 2]

            def group(g, _):
                for s16 in range(LANES):
                    s = g * LANES + s16
                    acc = ur[s, pl.ds(0, LANES)] * ir[s, pl.ds(0, LANES)]
                    for k in range(1, EMBED // LANES):
                        acc = acc + (ur[s, pl.ds(k * LANES, LANES)]
                                     * ir[s, pl.ds(k * LANES, LANES)])
                    plsc.store_scatter(tv, [lane * tpose + s16], acc)
                red = tv[pl.ds(0, LANES)]
                for l in range(1, LANES):
                    red = red + tv[pl.ds(l * tpose, LANES)]
                plsc.store_scatter(cfv, [j * CCH + g * LANES + lane], red)
                return 0

            lax.fori_loop(0, CCH // LANES, group, 0)

        cf_cp = cf_gather(0)
        g_cp = [None] * NCB
        s_cp = [None] * NCB
        for i in range(min(NCB - 2, nit)):
            g_cp[i % NCB] = cgather(i)
        for j in range(kc):
            nxt = cf_gather(j + 1) if j + 1 < kc else None
            for i in (2 * j, 2 * j + 1):
                g_cp[i % NCB].wait()
                s_cp[i % NCB] = cstore(i)
                ipre = i - 2
                inew = i + NCB - 2
                if inew < nit:
                    if ipre >= 0:
                        s_cp[ipre % NCB].wait()
                    g_cp[inew % NCB] = cgather(inew)
            cf_cp[0].wait()
            cf_cp[1].wait()
            cf_chunk(j)
            cf_cp = nxt
        cf_store = pltpu.async_copy(
            cfv, out_cf.at[pl.ds(wid * rows, rows)], cfsem)
        for i in range(max(0, nit - NCB), nit):
            s_cp[i % NCB].wait()
        cf_store.wait()

    return sc_kernel


def _tc_body(cf_ref, uc_ref, ic_ref, w1a_ref, w1b_ref, b1_ref,
             w2_ref, b2_ref, out_ref):
    bf = jnp.bfloat16
    h = jnp.dot(uc_ref[...].astype(bf), w1a_ref[...].astype(bf),
                preferred_element_type=jnp.float32)
    h = h + jnp.dot(ic_ref[...].astype(bf), w1b_ref[...].astype(bf),
                    preferred_element_type=jnp.float32)
    h = jnp.maximum(h + b1_ref[...], 0.0)
    out = jnp.dot(h.astype(bf), w2_ref[...].astype(bf),
                  preferred_element_type=jnp.float32)
    out_ref[...] = cf_ref[...][:, None] + out + b2_ref[...]


def _tc_mlp(cf, uc, ic, W1, b1, w2, b2, blk):
    batch = uc.shape[0]
    hid = w2.shape[0]
    grid = (batch // blk,)
    row_spec = pl.BlockSpec((blk, EMBED), lambda i: (i, 0))
    full = lambda shape: pl.BlockSpec(shape, lambda i: (0,) * len(shape))
    return pl.pallas_call(
        _tc_body,
        grid=grid,
        in_specs=[
            pl.BlockSpec((blk,), lambda i: (i,)),
            row_spec, row_spec,
            pl.BlockSpec((EMBED, hid), lambda i: (0, 0)),
            pl.BlockSpec((EMBED, hid), lambda i: (1, 0)),
            full((1, hid)),
            full((hid, EMBED)), full((1, EMBED)),
        ],
        out_specs=row_spec,
        out_shape=jax.ShapeDtypeStruct((batch, EMBED), jnp.float32),
    )(cf, uc, ic, W1, W1, b1, w2, b2)


def kernel(user_indices, item_indices, user_emb, item_emb,
           user_content_emb, item_content_emb, W1, b1, W2, b2):
    uidx = user_indices.astype(jnp.int32)
    iidx = item_indices.astype(jnp.int32)
    batch = uidx.shape[0]
    hid = W2.shape[0]
    b1r, b2r = b1.reshape(1, hid), b2.reshape(1, EMBED)

    sc = _make_sc_gather_cf(batch, user_emb.dtype)
    uc_g, ic_g, cf = sc(uidx, iidx, user_emb, item_emb,
                        user_content_emb, item_content_emb)
    return _tc_mlp(cf, uc_g, ic_g, W1, b1r, W2, b2r, blk=2048)


# speedup vs baseline: 1.1645x; 1.0422x over previous
import functools

import jax
import jax.numpy as jnp
from jax import lax
from jax.experimental import pallas as pl
from jax.experimental.pallas import tpu as pltpu
from jax.experimental.pallas import tpu_sc as plsc

EMBED = 128
CCH = 64
NCB = 6
LANES = 16


def _make_sc_gather_cf(batch, dtype):
    info = plsc.get_sparse_core_info()
    nc, ns = info.num_cores, info.num_subcores
    nw = nc * ns
    assert batch % (nw * CCH) == 0, (batch, nw)
    kc = batch // (nw * CCH)
    rows = kc * CCH
    nit = 2 * kc
    mesh = plsc.VectorSubcoreMesh(core_axis_name="c", subcore_axis_name="s")

    out_t = (
        jax.ShapeDtypeStruct((batch, EMBED), dtype),
        jax.ShapeDtypeStruct((batch, EMBED), dtype),
        jax.ShapeDtypeStruct((batch,), dtype),
    )

    scratch = (
        [pltpu.VMEM((rows,), jnp.int32),
         pltpu.VMEM((rows,), jnp.int32)]
        + [pltpu.VMEM((CCH, EMBED), dtype) for _ in range(4)]
        + [pltpu.VMEM((CCH, EMBED), dtype) for _ in range(NCB)]
        + [pltpu.VMEM((rows,), dtype),
           pltpu.VMEM((LANES * (LANES + 1),), dtype)]
        + [pltpu.SemaphoreType.DMA] * (4 + 2 * NCB + 1)
    )

    @functools.partial(
        pl.kernel,
        out_type=out_t,
        mesh=mesh,
        compiler_params=pltpu.CompilerParams(needs_layout_passes=False),
        scratch_types=scratch,
    )
    def sc_kernel(uidx_hbm, iidx_hbm, ue_hbm, ie_hbm, uc_hbm, ic_hbm,
                  out_uc, out_ic, out_cf, *scr):
        idx_u, idx_i = scr[0], scr[1]
        ue_slots, ie_slots = (scr[2], scr[3]), (scr[4], scr[5])
        cbufs = scr[6:6 + NCB]
        cfv, tv = scr[6 + NCB], scr[7 + NCB]
        sems = scr[8 + NCB:]
        usems, isems = (sems[0], sems[1]), (sems[2], sems[3])
        cgsems = sems[4:4 + NCB]
        cssems = sems[4 + NCB:4 + 2 * NCB]
        cfsem = sems[4 + 2 * NCB]

        wid = lax.axis_index("s") * nc + lax.axis_index("c")
        pltpu.sync_copy(uidx_hbm.at[pl.ds(wid * rows, rows)], idx_u)
        pltpu.sync_copy(iidx_hbm.at[pl.ds(wid * rows, rows)], idx_i)

        def cf_gather(j):
            b = j % 2
            sl = pl.ds(j * CCH, CCH)
            return (
                pltpu.async_copy(ue_hbm.at[idx_u.at[sl]], ue_slots[b], usems[b]),
                pltpu.async_copy(ie_hbm.at[idx_i.at[sl]], ie_slots[b], isems[b]),
            )

        def item(i):
            j, tbl = i // 2, i % 2
            table = uc_hbm if tbl == 0 else ic_hbm
            idxv = idx_u if tbl == 0 else idx_i
            out = out_uc if tbl == 0 else out_ic
            return table, idxv, j, out

        def cgather(i):
            table, idxv, j, _ = item(i)
            b = i % NCB
            return pltpu.async_copy(
                table.at[idxv.at[pl.ds(j * CCH, CCH)]], cbufs[b], cgsems[b])

        def cstore(i):
            _, _, j, out = item(i)
            b = i % NCB
            return pltpu.async_copy(
                cbufs[b], out.at[pl.ds(wid * rows + j * CCH, CCH)], cssems[b])

        lane = lax.iota(jnp.int32, LANES)
        tpose = LANES + 1

        def cf_chunk(j):
            ur, ir = ue_slots[j % 2], ie_slots[j % 2]

            def group(g, _):
                for s16 in range(LANES):
                    s = g * LANES + s16
                    acc = ur[s, pl.ds(0, LANES)] * ir[s, pl.ds(0, LANES)]
                    for k in range(1, EMBED // LANES):
                        acc = acc + (ur[s, pl.ds(k * LANES, LANES)]
                                     * ir[s, pl.ds(k * LANES, LANES)])
                    plsc.store_scatter(tv, [lane * tpose + s16], acc)
                red = tv[pl.ds(0, LANES)]
                for l in range(1, LANES):
                    red = red + tv[pl.ds(l * tpose, LANES)]
                plsc.store_scatter(cfv, [j * CCH + g * LANES + lane], red)
                return 0

            lax.fori_loop(0, CCH // LANES, group, 0)

        cf_cp = cf_gather(0)
        g_cp = [None] * NCB
        s_cp = [None] * NCB
        for i in range(min(NCB - 2, nit)):
            g_cp[i % NCB] = cgather(i)
        for j in range(kc):
            nxt = cf_gather(j + 1) if j + 1 < kc else None
            for i in (2 * j, 2 * j + 1):
                g_cp[i % NCB].wait()
                s_cp[i % NCB] = cstore(i)
                ipre = i - 2
                inew = i + NCB - 2
                if inew < nit:
                    if ipre >= 0:
                        s_cp[ipre % NCB].wait()
                    g_cp[inew % NCB] = cgather(inew)
            cf_cp[0].wait()
            cf_cp[1].wait()
            cf_chunk(j)
            cf_cp = nxt
        cf_store = pltpu.async_copy(
            cfv, out_cf.at[pl.ds(wid * rows, rows)], cfsem)
        for i in range(max(0, nit - NCB), nit):
            s_cp[i % NCB].wait()
        cf_store.wait()

    return sc_kernel


def _tc_body(cf_ref, uc_ref, ic_ref, w1a_ref, w1b_ref, b1_ref,
             w2_ref, b2_ref, out_ref):
    h = jnp.dot(uc_ref[...], w1a_ref[...], preferred_element_type=jnp.float32)
    h = h + jnp.dot(ic_ref[...], w1b_ref[...],
                    preferred_element_type=jnp.float32)
    h = jnp.maximum(h + b1_ref[...], 0.0)
    out = jnp.dot(h, w2_ref[...], preferred_element_type=jnp.float32)
    out_ref[...] = cf_ref[...][:, None] + out + b2_ref[...]


def _tc_mlp(cf, uc, ic, W1, b1, w2, b2, blk):
    batch = uc.shape[0]
    hid = w2.shape[0]
    grid = (batch // blk,)
    row_spec = pl.BlockSpec((blk, EMBED), lambda i: (i, 0))
    full = lambda shape: pl.BlockSpec(shape, lambda i: (0,) * len(shape))
    return pl.pallas_call(
        _tc_body,
        grid=grid,
        in_specs=[
            pl.BlockSpec((blk,), lambda i: (i,)),
            row_spec, row_spec,
            pl.BlockSpec((EMBED, hid), lambda i: (0, 0)),
            pl.BlockSpec((EMBED, hid), lambda i: (1, 0)),
            full((1, hid)),
            full((hid, EMBED)), full((1, EMBED)),
        ],
        out_specs=row_spec,
        out_shape=jax.ShapeDtypeStruct((batch, EMBED), jnp.float32),
    )(cf, uc, ic, W1, W1, b1, w2, b2)


def kernel(user_indices, item_indices, user_emb, item_emb,
           user_content_emb, item_content_emb, W1, b1, W2, b2):
    uidx = user_indices.astype(jnp.int32)
    iidx = item_indices.astype(jnp.int32)
    batch = uidx.shape[0]
    hid = W2.shape[0]
    b1r, b2r = b1.reshape(1, hid), b2.reshape(1, EMBED)

    sc = _make_sc_gather_cf(batch, user_emb.dtype)
    uc_g, ic_g, cf = sc(uidx, iidx, user_emb, item_emb,
                        user_content_emb, item_content_emb)
    return _tc_mlp(cf, uc_g, ic_g, W1, b1r, W2, b2r, blk=4096)


# speedup vs baseline: 1.1798x; 1.0131x over previous
import functools

import jax
import jax.numpy as jnp
from jax import lax
from jax.experimental import pallas as pl
from jax.experimental.pallas import tpu as pltpu
from jax.experimental.pallas import tpu_sc as plsc

EMBED = 128
CCH = 64
NCB = 6
LANES = 16


def _make_sc_gather_cf(batch, dtype):
    info = plsc.get_sparse_core_info()
    nc, ns = info.num_cores, info.num_subcores
    nw = nc * ns
    assert batch % (nw * CCH) == 0, (batch, nw)
    kc = batch // (nw * CCH)
    rows = kc * CCH
    nit = 2 * kc
    mesh = plsc.VectorSubcoreMesh(core_axis_name="c", subcore_axis_name="s")

    out_t = (
        jax.ShapeDtypeStruct((batch, EMBED), dtype),
        jax.ShapeDtypeStruct((batch, EMBED), dtype),
        jax.ShapeDtypeStruct((batch,), dtype),
    )

    scratch = (
        [pltpu.VMEM((rows,), jnp.int32),
         pltpu.VMEM((rows,), jnp.int32)]
        + [pltpu.VMEM((CCH, EMBED), dtype) for _ in range(4)]
        + [pltpu.VMEM((CCH, EMBED), dtype) for _ in range(NCB)]
        + [pltpu.VMEM((rows,), dtype),
           pltpu.VMEM((LANES * (LANES + 1),), dtype)]
        + [pltpu.SemaphoreType.DMA] * (4 + 2 * NCB + 1)
    )

    @functools.partial(
        pl.kernel,
        out_type=out_t,
        mesh=mesh,
        compiler_params=pltpu.CompilerParams(needs_layout_passes=False),
        scratch_types=scratch,
    )
    def sc_kernel(uidx_hbm, iidx_hbm, ue_hbm, ie_hbm, uc_hbm, ic_hbm,
                  out_uc, out_ic, out_cf, *scr):
        idx_u, idx_i = scr[0], scr[1]
        ue_slots, ie_slots = (scr[2], scr[3]), (scr[4], scr[5])
        cbufs = scr[6:6 + NCB]
        cfv, tv = scr[6 + NCB], scr[7 + NCB]
        sems = scr[8 + NCB:]
        usems, isems = (sems[0], sems[1]), (sems[2], sems[3])
        cgsems = sems[4:4 + NCB]
        cssems = sems[4 + NCB:4 + 2 * NCB]
        cfsem = sems[4 + 2 * NCB]

        wid = lax.axis_index("s") * nc + lax.axis_index("c")
        pltpu.sync_copy(uidx_hbm.at[pl.ds(wid * rows, rows)], idx_u)
        pltpu.sync_copy(iidx_hbm.at[pl.ds(wid * rows, rows)], idx_i)

        def cf_gather(j):
            b = j % 2
            sl = pl.ds(j * CCH, CCH)
            return (
                pltpu.async_copy(ue_hbm.at[idx_u.at[sl]], ue_slots[b], usems[b]),
                pltpu.async_copy(ie_hbm.at[idx_i.at[sl]], ie_slots[b], isems[b]),
            )

        def item(i):
            j, tbl = i // 2, i % 2
            table = uc_hbm if tbl == 0 else ic_hbm
            idxv = idx_u if tbl == 0 else idx_i
            out = out_uc if tbl == 0 else out_ic
            return table, idxv, j, out

        def cgather(i):
            table, idxv, j, _ = item(i)
            b = i % NCB
            return pltpu.async_copy(
                table.at[idxv.at[pl.ds(j * CCH, CCH)]], cbufs[b], cgsems[b])

        def cstore(i):
            _, _, j, out = item(i)
            b = i % NCB
            return pltpu.async_copy(
                cbufs[b], out.at[pl.ds(wid * rows + j * CCH, CCH)], cssems[b])

        lane = lax.iota(jnp.int32, LANES)
        tpose = LANES + 1

        def cf_chunk(j):
            ur, ir = ue_slots[j % 2], ie_slots[j % 2]

            def group(g, _):
                for s16 in range(LANES):
                    s = g * LANES + s16
                    acc = ur[s, pl.ds(0, LANES)] * ir[s, pl.ds(0, LANES)]
                    for k in range(1, EMBED // LANES):
                        acc = acc + (ur[s, pl.ds(k * LANES, LANES)]
                                     * ir[s, pl.ds(k * LANES, LANES)])
                    plsc.store_scatter(tv, [lane * tpose + s16], acc)
                red = tv[pl.ds(0, LANES)]
                for l in range(1, LANES):
                    red = red + tv[pl.ds(l * tpose, LANES)]
                plsc.store_scatter(cfv, [j * CCH + g * LANES + lane], red)
                return 0

            lax.fori_loop(0, CCH // LANES, group, 0)

        cf_cp = cf_gather(0)
        g_cp = [None] * NCB
        s_cp = [None] * NCB
        for i in range(min(NCB - 2, nit)):
            g_cp[i % NCB] = cgather(i)
        for j in range(kc):
            nxt = cf_gather(j + 1) if j + 1 < kc else None
            for i in (2 * j, 2 * j + 1):
                g_cp[i % NCB].wait()
                s_cp[i % NCB] = cstore(i)
                ipre = i - 2
                inew = i + NCB - 2
                if inew < nit:
                    if ipre >= 0:
                        s_cp[ipre % NCB].wait()
                    g_cp[inew % NCB] = cgather(inew)
            cf_cp[0].wait()
            cf_cp[1].wait()
            cf_chunk(j)
            cf_cp = nxt
        cf_store = pltpu.async_copy(
            cfv, out_cf.at[pl.ds(wid * rows, rows)], cfsem)
        for i in range(max(0, nit - NCB), nit):
            s_cp[i % NCB].wait()
        cf_store.wait()

    return sc_kernel


def _tc_body(cf_ref, uc_ref, ic_ref, w1a_ref, w1b_ref, b1_ref,
             w2_ref, b2_ref, out_ref):
    h = jnp.dot(uc_ref[...], w1a_ref[...], preferred_element_type=jnp.float32)
    h = h + jnp.dot(ic_ref[...], w1b_ref[...],
                    preferred_element_type=jnp.float32)
    h = jnp.maximum(h + b1_ref[...], 0.0)
    out = jnp.dot(h, w2_ref[...], preferred_element_type=jnp.float32)
    out_ref[...] = cf_ref[...][:, None] + out + b2_ref[...]


def _tc_mlp(cf, uc, ic, W1, b1, w2, b2, blk):
    batch = uc.shape[0]
    hid = w2.shape[0]
    grid = (batch // blk,)
    row_spec = pl.BlockSpec((blk, EMBED), lambda i: (i, 0))
    full = lambda shape: pl.BlockSpec(shape, lambda i: (0,) * len(shape))
    return pl.pallas_call(
        _tc_body,
        grid=grid,
        in_specs=[
            pl.BlockSpec((blk,), lambda i: (i,)),
            row_spec, row_spec,
            pl.BlockSpec((EMBED, hid), lambda i: (0, 0)),
            pl.BlockSpec((EMBED, hid), lambda i: (1, 0)),
            full((1, hid)),
            full((hid, EMBED)), full((1, EMBED)),
        ],
        out_specs=row_spec,
        out_shape=jax.ShapeDtypeStruct((batch, EMBED), jnp.float32),
    )(cf, uc, ic, W1, W1, b1, w2, b2)


def kernel(user_indices, item_indices, user_emb, item_emb,
           user_content_emb, item_content_emb, W1, b1, W2, b2):
    uidx = user_indices.astype(jnp.int32)
    iidx = item_indices.astype(jnp.int32)
    batch = uidx.shape[0]
    hid = W2.shape[0]
    b1r, b2r = b1.reshape(1, hid), b2.reshape(1, EMBED)

    sc = _make_sc_gather_cf(batch, user_emb.dtype)
    uc_g, ic_g, cf = sc(uidx, iidx, user_emb, item_emb,
                        user_content_emb, item_content_emb)
    return _tc_mlp(cf, uc_g, ic_g, W1, b1r, W2, b2r, blk=8192)
